# baseline (speedup 1.0000x reference)
import functools

import jax
import jax.numpy as jnp
from jax import lax
from jax.experimental import pallas as pl
from jax.experimental.pallas import tpu as pltpu
from jax.experimental.pallas import tpu_sc as plsc

B, N, S, D1, D2 = 4, 8192, 2048, 128, 256
C0, C1 = 256, 128
BN_COUNT = B * N


TN1 = 512


def _topk_body(x1_ref, x2_ref, idx_ref, w_ref):
    b = pl.program_id(0)
    x1 = x1_ref[0]
    x2 = x2_ref[0]
    n1 = jnp.sum(x1 * x1, axis=1, keepdims=True)
    n2 = jnp.sum(x2 * x2, axis=0, keepdims=True)
    xy = lax.dot_general(x1.astype(jnp.bfloat16), x2.astype(jnp.bfloat16),
                         (((1,), (0,)), ((), ())),
                         preferred_element_type=jnp.float32)
    d = (-2.0 * xy + n1) + n2

    iota = lax.broadcasted_iota(jnp.int32, (TN1, S), 1)
    big = jnp.float32(jnp.inf)
    ds_ = []
    is_ = []
    for _ in range(3):
        m = jnp.min(d, axis=1, keepdims=True)
        i = jnp.min(jnp.where(d == m, iota, S), axis=1, keepdims=True)
        ds_.append(m)
        is_.append(i)
        d = jnp.where(iota == i, big, d)
    d3 = jnp.concatenate(ds_, axis=1)
    i3 = jnp.concatenate(is_, axis=1)
    d3 = jnp.maximum(d3, 1e-10)
    recip = 1.0 / d3
    w = recip / jnp.sum(recip, axis=1, keepdims=True)

    idx_ref[0] = i3 + b * S
    w_ref[0] = jnp.concatenate(
        [jnp.broadcast_to(w[:, k:k + 1], (TN1, 16)) for k in range(3)], axis=1)


def _run_topk(xyz1t, xyz2):
    grid = (B, N // TN1)
    return pl.pallas_call(
        _topk_body,
        grid=grid,
        in_specs=[
            pl.BlockSpec((1, TN1, 3), lambda b, n: (b, n, 0)),
            pl.BlockSpec((1, 3, S), lambda b, n: (b, 0, 0)),
        ],
        out_specs=[
            pl.BlockSpec((1, TN1, 3), lambda b, n: (b, n, 0)),
            pl.BlockSpec((1, TN1, 48), lambda b, n: (b, n, 0)),
        ],
        out_shape=[
            jax.ShapeDtypeStruct((B, N, 3), jnp.int32),
            jax.ShapeDtypeStruct((B, N, 48), jnp.float32),
        ],
    )(xyz1t, xyz2)


SC_Q = 32
SC_NW = 32
SC_PER_W = BN_COUNT // SC_NW


def _sc_gather_body(table, idx3, wcat, out,
                    idx_v0, idx_v1, w_v0, w_v1, rows_v0, rows_v1, out_v,
                    s_i0, s_i1, s_w0, s_w1, s_g0, s_g1):
    wid = lax.axis_index("s") * 2 + lax.axis_index("c")
    idx_v = (idx_v0, idx_v1)
    w_v = (w_v0, w_v1)
    rows_v = (rows_v0, rows_v1)
    s_i = (s_i0, s_i1)
    s_w = (s_w0, s_w1)
    s_g = (s_g0, s_g1)
    nsteps = SC_PER_W // SC_Q

    def qbase(j):
        return wid * SC_PER_W + j * SC_Q

    def start_a(j, slot):
        pltpu.async_copy(idx3.at[pl.ds(qbase(j) * 3, 3 * SC_Q)],
                         idx_v[slot], s_i[slot])
        pltpu.async_copy(wcat.at[pl.ds(qbase(j), SC_Q)], w_v[slot], s_w[slot])

    def wait_a(j, slot):
        pltpu.make_async_copy(idx3.at[pl.ds(qbase(j) * 3, 3 * SC_Q)],
                              idx_v[slot], s_i[slot]).wait()
        pltpu.make_async_copy(wcat.at[pl.ds(qbase(j), SC_Q)],
                              w_v[slot], s_w[slot]).wait()

    def start_b(slot):
        pltpu.async_copy(table.at[idx_v[slot]], rows_v[slot], s_g[slot])

    def wait_b(slot):
        pltpu.make_async_copy(table.at[idx_v[slot]], rows_v[slot],
                              s_g[slot]).wait()

    def compute(j, slot):
        rv = rows_v[slot]
        wv = w_v[slot]
        for r in range(SC_Q):
            w0 = wv[r, 0:16]
            w1 = wv[r, 16:32]
            w2 = wv[r, 32:48]
            for c in range(D2 // 16):
                sl = pl.ds(c * 16, 16)
                out_v[r, sl] = (w0 * rv[3 * r, sl]
                                + w1 * rv[3 * r + 1, sl]
                                + w2 * rv[3 * r + 2, sl])
        pltpu.sync_copy(out_v, out.at[pl.ds(qbase(j), SC_Q)])

    start_a(0, 0)
    start_a(1, 1)
    wait_a(0, 0)
    start_b(0)

    def iteration(i, carry):
        j0 = 2 * i
        j1 = j0 + 1
        wait_a(j1, 1)
        wait_b(0)
        start_b(1)
        compute(j0, 0)

        @pl.when(j1 + 1 < nsteps)
        def _():
            start_a(j1 + 1, 0)
        wait_b(1)
        compute(j1, 1)

        @pl.when(j1 + 2 < nsteps)
        def _():
            start_a(j1 + 2, 1)

        @pl.when(j1 + 1 < nsteps)
        def _():
            wait_a(j1 + 1, 0)
            start_b(0)
        return carry

    lax.fori_loop(0, nsteps // 2, iteration, 0)


def _run_sc_gather(table, idx3, wcat):
    mesh = plsc.VectorSubcoreMesh(core_axis_name="c", subcore_axis_name="s")
    fn = pl.kernel(
        _sc_gather_body,
        out_type=jax.ShapeDtypeStruct((BN_COUNT, D2), jnp.float32),
        mesh=mesh,
        scratch_types=[
            pltpu.VMEM((3 * SC_Q,), jnp.int32),
            pltpu.VMEM((3 * SC_Q,), jnp.int32),
            pltpu.VMEM((SC_Q, 48), jnp.float32),
            pltpu.VMEM((SC_Q, 48), jnp.float32),
            pltpu.VMEM((3 * SC_Q, D2), jnp.float32),
            pltpu.VMEM((3 * SC_Q, D2), jnp.float32),
            pltpu.VMEM((SC_Q, D2), jnp.float32),
            pltpu.SemaphoreType.DMA,
            pltpu.SemaphoreType.DMA,
            pltpu.SemaphoreType.DMA,
            pltpu.SemaphoreType.DMA,
            pltpu.SemaphoreType.DMA,
            pltpu.SemaphoreType.DMA,
        ],
    )
    return fn(table, idx3, wcat)


TNM = 512


def _layer0_body(p1_ref, it_ref, wa_ref, wb_ref, b_ref, y_ref, st_ref):
    step = pl.program_id(0) * pl.num_programs(1) + pl.program_id(1)
    p1 = p1_ref[0]
    it = it_ref[0]
    y = (lax.dot_general(p1, wa_ref[...], (((0,), (0,)), ((), ())),
                         preferred_element_type=jnp.float32)
         + jnp.dot(it, wb_ref[...], preferred_element_type=jnp.float32)
         + b_ref[...])
    y_ref[0] = y

    @pl.when(step == 0)
    def _():
        st_ref[...] = jnp.zeros_like(st_ref)

    s0 = jnp.sum(y, axis=0, keepdims=True)
    s1 = jnp.sum(y * y, axis=0, keepdims=True)
    st_ref[0:1, :] += s0
    st_ref[1:2, :] += s1


def _run_layer0(p1t, interp, wa, wb, b0r):
    grid = (B, N // TNM)
    return pl.pallas_call(
        _layer0_body,
        grid=grid,
        in_specs=[
            pl.BlockSpec((1, D1, TNM), lambda b, n: (b, 0, n)),
            pl.BlockSpec((1, TNM, D2), lambda b, n: (b, n, 0)),
            pl.BlockSpec((D1, C0), lambda b, n: (0, 0)),
            pl.BlockSpec((D2, C0), lambda b, n: (0, 0)),
            pl.BlockSpec((1, C0), lambda b, n: (0, 0)),
        ],
        out_specs=[
            pl.BlockSpec((1, TNM, C0), lambda b, n: (b, n, 0)),
            pl.BlockSpec((8, C0), lambda b, n: (0, 0)),
        ],
        out_shape=[
            jax.ShapeDtypeStruct((B, N, C0), jnp.float32),
            jax.ShapeDtypeStruct((8, C0), jnp.float32),
        ],
    )(p1t, interp, wa, wb, b0r)


def _layer1_body(y0_ref, a_ref, c_ref, w_ref, b_ref, y_ref, st_ref):
    step = pl.program_id(0) * pl.num_programs(1) + pl.program_id(1)
    h = jnp.maximum(y0_ref[0] * a_ref[...] + c_ref[...], 0.0)
    y = jnp.dot(h, w_ref[...], preferred_element_type=jnp.float32) + b_ref[...]
    y_ref[0] = y

    @pl.when(step == 0)
    def _():
        st_ref[...] = jnp.zeros_like(st_ref)

    st_ref[0:1, :] += jnp.sum(y, axis=0, keepdims=True)
    st_ref[1:2, :] += jnp.sum(y * y, axis=0, keepdims=True)


def _run_layer1(y0t, a0, c0, w1t, b1r):
    grid = (B, N // TNM)
    return pl.pallas_call(
        _layer1_body,
        grid=grid,
        in_specs=[
            pl.BlockSpec((1, TNM, C0), lambda b, n: (b, n, 0)),
            pl.BlockSpec((1, C0), lambda b, n: (0, 0)),
            pl.BlockSpec((1, C0), lambda b, n: (0, 0)),
            pl.BlockSpec((C0, C1), lambda b, n: (0, 0)),
            pl.BlockSpec((1, C1), lambda b, n: (0, 0)),
        ],
        out_specs=[
            pl.BlockSpec((1, TNM, C1), lambda b, n: (b, n, 0)),
            pl.BlockSpec((8, C1), lambda b, n: (0, 0)),
        ],
        out_shape=[
            jax.ShapeDtypeStruct((B, N, C1), jnp.float32),
            jax.ShapeDtypeStruct((8, C1), jnp.float32),
        ],
    )(y0t, a0, c0, w1t, b1r)


def _final_body(y1_ref, a_ref, c_ref, o_ref):
    res = jnp.maximum(y1_ref[0] * a_ref[...] + c_ref[...], 0.0)
    o_ref[0] = jnp.transpose(res)


def _run_final(y1t, a1, c1):
    grid = (B, N // TNM)
    return pl.pallas_call(
        _final_body,
        grid=grid,
        in_specs=[
            pl.BlockSpec((1, TNM, C1), lambda b, n: (b, n, 0)),
            pl.BlockSpec((1, C1), lambda b, n: (0, 0)),
            pl.BlockSpec((1, C1), lambda b, n: (0, 0)),
        ],
        out_specs=pl.BlockSpec((1, C1, TNM), lambda b, n: (b, 0, n)),
        out_shape=jax.ShapeDtypeStruct((B, C1, N), jnp.float32),
    )(y1t, a1, c1)


def _bn_coeffs(stats, g, beta):
    mean = stats[0, :] / BN_COUNT
    var = stats[1, :] / BN_COUNT - mean * mean
    a = g / jnp.sqrt(var + 1e-5)
    c = beta - mean * a
    return a[None, :], c[None, :]


@jax.jit
def kernel(xyz1, xyz2, points1, points2, W0, b0, g0, beta0, W1, b1, g1, beta1):
    xyz1t = jnp.transpose(xyz1, (0, 2, 1))
    table = jnp.transpose(points2, (0, 2, 1)).reshape(B * S, D2)

    idx, wcat = _run_topk(xyz1t, xyz2)
    idx3 = idx.reshape(B * N * 3)
    wf = wcat.reshape(B * N, 48)

    interp = _run_sc_gather(table, idx3, wf)
    interp = interp.reshape(B, N, D2)

    wa = jnp.transpose(W0[:, :D1])
    wb = jnp.transpose(W0[:, D1:])
    y0t, st0 = _run_layer0(points1, interp, wa, wb, b0[None, :])
    a0, c0 = _bn_coeffs(st0, g0, beta0)

    y1t, st1 = _run_layer1(y0t, a0, c0, jnp.transpose(W1), b1[None, :])
    a1, c1 = _bn_coeffs(st1, g1, beta1)

    return _run_final(y1t, a1, c1)


# speedup vs baseline: 1.0746x; 1.0746x over previous
import functools

import jax
import jax.numpy as jnp
from jax import lax
from jax.experimental import pallas as pl
from jax.experimental.pallas import tpu as pltpu
from jax.experimental.pallas import tpu_sc as plsc

B, N, S, D1, D2 = 4, 8192, 2048, 128, 256
C0, C1 = 256, 128
BN_COUNT = B * N


TN1 = 512


def _topk_body(x1_ref, x2_ref, idx_ref, w_ref):
    b = pl.program_id(0)
    x1 = x1_ref[0]
    x2 = x2_ref[0]
    n1 = jnp.sum(x1 * x1, axis=1, keepdims=True)
    n2 = jnp.sum(x2 * x2, axis=0, keepdims=True)
    xy = lax.dot_general(x1.astype(jnp.bfloat16), x2.astype(jnp.bfloat16),
                         (((1,), (0,)), ((), ())),
                         preferred_element_type=jnp.float32)
    d = (-2.0 * xy + n1) + n2

    iota = lax.broadcasted_iota(jnp.int32, (TN1, S), 1)
    big = jnp.float32(jnp.inf)
    ds_ = []
    is_ = []
    for _ in range(3):
        m = jnp.min(d, axis=1, keepdims=True)
        i = jnp.min(jnp.where(d == m, iota, S), axis=1, keepdims=True)
        ds_.append(m)
        is_.append(i)
        d = jnp.where(iota == i, big, d)
    d3 = jnp.concatenate(ds_, axis=1)
    i3 = jnp.concatenate(is_, axis=1)
    d3 = jnp.maximum(d3, 1e-10)
    recip = 1.0 / d3
    w = recip / jnp.sum(recip, axis=1, keepdims=True)

    idx_ref[0] = i3 + b * S
    w_ref[0] = jnp.concatenate(
        [jnp.broadcast_to(w[:, k:k + 1], (TN1, 16)) for k in range(3)], axis=1)


def _run_topk(xyz1t, xyz2):
    grid = (B, N // TN1)
    return pl.pallas_call(
        _topk_body,
        grid=grid,
        in_specs=[
            pl.BlockSpec((1, TN1, 3), lambda b, n: (b, n, 0)),
            pl.BlockSpec((1, 3, S), lambda b, n: (b, 0, 0)),
        ],
        out_specs=[
            pl.BlockSpec((1, TN1, 3), lambda b, n: (b, n, 0)),
            pl.BlockSpec((1, TN1, 48), lambda b, n: (b, n, 0)),
        ],
        out_shape=[
            jax.ShapeDtypeStruct((B, N, 3), jnp.int32),
            jax.ShapeDtypeStruct((B, N, 48), jnp.float32),
        ],
    )(xyz1t, xyz2)


SC_Q = 16
SC_NW = 32
SC_PER_W = BN_COUNT // SC_NW


def _sc_gather_body(table, idx3, wcat, out,
                    idx_vs, w_vs, rows_vs, out_vs, s_is, s_ws, s_gs, s_os):
    wid = lax.axis_index("s") * 2 + lax.axis_index("c")
    nsteps = SC_PER_W // SC_Q

    def qbase(j):
        return wid * SC_PER_W + j * SC_Q

    def start_a(j, slot):
        pltpu.async_copy(idx3.at[pl.ds(qbase(j) * 3, 3 * SC_Q)],
                         idx_vs[slot], s_is[slot])
        pltpu.async_copy(wcat.at[pl.ds(qbase(j), SC_Q)], w_vs[slot], s_ws[slot])

    def wait_a(j, slot):
        pltpu.make_async_copy(idx3.at[pl.ds(qbase(j) * 3, 3 * SC_Q)],
                              idx_vs[slot], s_is[slot]).wait()
        pltpu.make_async_copy(wcat.at[pl.ds(qbase(j), SC_Q)],
                              w_vs[slot], s_ws[slot]).wait()

    def start_b(slot):
        pltpu.async_copy(table.at[idx_vs[slot]], rows_vs[slot], s_gs[slot])

    def wait_b(slot):
        pltpu.make_async_copy(table.at[idx_vs[slot]], rows_vs[slot],
                              s_gs[slot]).wait()

    def start_o(j, oslot):
        pltpu.async_copy(out_vs[oslot], out.at[pl.ds(qbase(j), SC_Q)],
                         s_os[oslot])

    def wait_o(j, oslot):
        pltpu.make_async_copy(out_vs[oslot], out.at[pl.ds(qbase(j), SC_Q)],
                              s_os[oslot]).wait()

    def compute(j, slot, oslot):
        rv = rows_vs[slot]
        wv = w_vs[slot]
        ov = out_vs[oslot]
        for r in range(SC_Q):
            w0 = wv[r, 0:16]
            w1 = wv[r, 16:32]
            w2 = wv[r, 32:48]
            for c in range(D2 // 16):
                sl = pl.ds(c * 16, 16)
                ov[r, sl] = (w0 * rv[3 * r, sl]
                             + w1 * rv[3 * r + 1, sl]
                             + w2 * rv[3 * r + 2, sl])
        start_o(j, oslot)

    for t in range(4):
        start_a(t, t)
    wait_a(0, 0)
    start_b(0)
    wait_a(1, 1)
    start_b(1)

    def iteration(i, carry):
        j = 4 * i
        for t in range(4):
            jt = j + t
            wait_b(t)
            g = jt + 2

            @pl.when(g < nsteps)
            def _(g=g, t=t):
                wait_a(g, (t + 2) % 4)
                start_b((t + 2) % 4)

            @pl.when(jt >= 2)
            def _(jt=jt, t=t):
                wait_o(jt - 2, t % 2)
            compute(jt, t, t % 2)
            p = jt + 4

            @pl.when(p < nsteps)
            def _(p=p, t=t):
                start_a(p, t)
        return carry

    lax.fori_loop(0, nsteps // 4, iteration, 0)
    wait_o(nsteps - 2, 0)
    wait_o(nsteps - 1, 1)


def _run_sc_gather(table, idx3, wcat):
    mesh = plsc.VectorSubcoreMesh(core_axis_name="c", subcore_axis_name="s")
    fn = pl.kernel(
        _sc_gather_body,
        out_type=jax.ShapeDtypeStruct((BN_COUNT, D2), jnp.float32),
        mesh=mesh,
        scratch_types=[
            [pltpu.VMEM((3 * SC_Q,), jnp.int32) for _ in range(4)],
            [pltpu.VMEM((SC_Q, 48), jnp.float32) for _ in range(4)],
            [pltpu.VMEM((3 * SC_Q, D2), jnp.float32) for _ in range(4)],
            [pltpu.VMEM((SC_Q, D2), jnp.float32) for _ in range(2)],
            [pltpu.SemaphoreType.DMA for _ in range(4)],
            [pltpu.SemaphoreType.DMA for _ in range(4)],
            [pltpu.SemaphoreType.DMA for _ in range(4)],
            [pltpu.SemaphoreType.DMA for _ in range(2)],
        ],
    )
    return fn(table, idx3, wcat)


TNM = 512


def _layer0_body(p1_ref, it_ref, wa_ref, wb_ref, b_ref, y_ref, st_ref):
    step = pl.program_id(0) * pl.num_programs(1) + pl.program_id(1)
    p1 = p1_ref[0]
    it = it_ref[0]
    y = (lax.dot_general(p1, wa_ref[...], (((0,), (0,)), ((), ())),
                         preferred_element_type=jnp.float32)
         + jnp.dot(it, wb_ref[...], preferred_element_type=jnp.float32)
         + b_ref[...])
    y_ref[0] = y

    @pl.when(step == 0)
    def _():
        st_ref[...] = jnp.zeros_like(st_ref)

    s0 = jnp.sum(y, axis=0, keepdims=True)
    s1 = jnp.sum(y * y, axis=0, keepdims=True)
    st_ref[0:1, :] += s0
    st_ref[1:2, :] += s1


def _run_layer0(p1t, interp, wa, wb, b0r):
    grid = (B, N // TNM)
    return pl.pallas_call(
        _layer0_body,
        grid=grid,
        in_specs=[
            pl.BlockSpec((1, D1, TNM), lambda b, n: (b, 0, n)),
            pl.BlockSpec((1, TNM, D2), lambda b, n: (b, n, 0)),
            pl.BlockSpec((D1, C0), lambda b, n: (0, 0)),
            pl.BlockSpec((D2, C0), lambda b, n: (0, 0)),
            pl.BlockSpec((1, C0), lambda b, n: (0, 0)),
        ],
        out_specs=[
            pl.BlockSpec((1, TNM, C0), lambda b, n: (b, n, 0)),
            pl.BlockSpec((8, C0), lambda b, n: (0, 0)),
        ],
        out_shape=[
            jax.ShapeDtypeStruct((B, N, C0), jnp.float32),
            jax.ShapeDtypeStruct((8, C0), jnp.float32),
        ],
    )(p1t, interp, wa, wb, b0r)


def _layer1_body(y0_ref, a_ref, c_ref, w_ref, b_ref, y_ref, st_ref):
    step = pl.program_id(0) * pl.num_programs(1) + pl.program_id(1)
    h = jnp.maximum(y0_ref[0] * a_ref[...] + c_ref[...], 0.0)
    y = jnp.dot(h, w_ref[...], preferred_element_type=jnp.float32) + b_ref[...]
    y_ref[0] = y

    @pl.when(step == 0)
    def _():
        st_ref[...] = jnp.zeros_like(st_ref)

    st_ref[0:1, :] += jnp.sum(y, axis=0, keepdims=True)
    st_ref[1:2, :] += jnp.sum(y * y, axis=0, keepdims=True)


def _run_layer1(y0t, a0, c0, w1t, b1r):
    grid = (B, N // TNM)
    return pl.pallas_call(
        _layer1_body,
        grid=grid,
        in_specs=[
            pl.BlockSpec((1, TNM, C0), lambda b, n: (b, n, 0)),
            pl.BlockSpec((1, C0), lambda b, n: (0, 0)),
            pl.BlockSpec((1, C0), lambda b, n: (0, 0)),
            pl.BlockSpec((C0, C1), lambda b, n: (0, 0)),
            pl.BlockSpec((1, C1), lambda b, n: (0, 0)),
        ],
        out_specs=[
            pl.BlockSpec((1, TNM, C1), lambda b, n: (b, n, 0)),
            pl.BlockSpec((8, C1), lambda b, n: (0, 0)),
        ],
        out_shape=[
            jax.ShapeDtypeStruct((B, N, C1), jnp.float32),
            jax.ShapeDtypeStruct((8, C1), jnp.float32),
        ],
    )(y0t, a0, c0, w1t, b1r)


def _final_body(y1_ref, a_ref, c_ref, o_ref):
    res = jnp.maximum(y1_ref[0] * a_ref[...] + c_ref[...], 0.0)
    o_ref[0] = jnp.transpose(res)


def _run_final(y1t, a1, c1):
    grid = (B, N // TNM)
    return pl.pallas_call(
        _final_body,
        grid=grid,
        in_specs=[
            pl.BlockSpec((1, TNM, C1), lambda b, n: (b, n, 0)),
            pl.BlockSpec((1, C1), lambda b, n: (0, 0)),
            pl.BlockSpec((1, C1), lambda b, n: (0, 0)),
        ],
        out_specs=pl.BlockSpec((1, C1, TNM), lambda b, n: (b, 0, n)),
        out_shape=jax.ShapeDtypeStruct((B, C1, N), jnp.float32),
    )(y1t, a1, c1)


def _bn_coeffs(stats, g, beta):
    mean = stats[0, :] / BN_COUNT
    var = stats[1, :] / BN_COUNT - mean * mean
    a = g / jnp.sqrt(var + 1e-5)
    c = beta - mean * a
    return a[None, :], c[None, :]


@jax.jit
def kernel(xyz1, xyz2, points1, points2, W0, b0, g0, beta0, W1, b1, g1, beta1):
    xyz1t = jnp.transpose(xyz1, (0, 2, 1))
    table = jnp.transpose(points2, (0, 2, 1)).reshape(B * S, D2)

    idx, wcat = _run_topk(xyz1t, xyz2)
    idx3 = idx.reshape(B * N * 3)
    wf = wcat.reshape(B * N, 48)

    interp = _run_sc_gather(table, idx3, wf)
    interp = interp.reshape(B, N, D2)

    wa = jnp.transpose(W0[:, :D1])
    wb = jnp.transpose(W0[:, D1:])
    y0t, st0 = _run_layer0(points1, interp, wa, wb, b0[None, :])
    a0, c0 = _bn_coeffs(st0, g0, beta0)

    y1t, st1 = _run_layer1(y0t, a0, c0, jnp.transpose(W1), b1[None, :])
    a1, c1 = _bn_coeffs(st1, g1, beta1)

    return _run_final(y1t, a1, c1)


# speedup vs baseline: 1.2444x; 1.1581x over previous
import functools

import jax
import jax.numpy as jnp
from jax import lax
from jax.experimental import pallas as pl
from jax.experimental.pallas import tpu as pltpu
from jax.experimental.pallas import tpu_sc as plsc

B, N, S, D1, D2 = 4, 8192, 2048, 128, 256
C0, C1 = 256, 128
BN_COUNT = B * N


TN1 = 512


def _topk_body(boff, x1_ref, x2_ref, idx_ref, w_ref):
    x1 = x1_ref[0]
    x2 = x2_ref[0]
    n1 = jnp.sum(x1 * x1, axis=1, keepdims=True)
    n2 = jnp.sum(x2 * x2, axis=0, keepdims=True)
    xy = lax.dot_general(x1.astype(jnp.bfloat16), x2.astype(jnp.bfloat16),
                         (((1,), (0,)), ((), ())),
                         preferred_element_type=jnp.float32)
    d = (-2.0 * xy + n1) + n2

    iota = lax.broadcasted_iota(jnp.int32, (TN1, S), 1)
    big = jnp.float32(jnp.inf)
    ds_ = []
    is_ = []
    for _ in range(3):
        m = jnp.min(d, axis=1, keepdims=True)
        i = jnp.min(jnp.where(d == m, iota, S), axis=1, keepdims=True)
        ds_.append(m)
        is_.append(i)
        d = jnp.where(iota == i, big, d)
    d3 = jnp.concatenate(ds_, axis=1)
    i3 = jnp.concatenate(is_, axis=1)
    d3 = jnp.maximum(d3, 1e-10)
    recip = 1.0 / d3
    w = recip / jnp.sum(recip, axis=1, keepdims=True)

    idx_ref[0] = i3 + boff
    w_ref[0] = jnp.concatenate(
        [jnp.broadcast_to(w[:, k:k + 1], (TN1, 16)) for k in range(3)], axis=1)


def _run_topk(xyz1t_b, xyz2_b, boff):
    return pl.pallas_call(
        functools.partial(_topk_body, boff),
        grid=(N // TN1,),
        in_specs=[
            pl.BlockSpec((1, TN1, 3), lambda n: (0, n, 0)),
            pl.BlockSpec((1, 3, S), lambda n: (0, 0, 0)),
        ],
        out_specs=[
            pl.BlockSpec((1, TN1, 3), lambda n: (0, n, 0)),
            pl.BlockSpec((1, TN1, 48), lambda n: (0, n, 0)),
        ],
        out_shape=[
            jax.ShapeDtypeStruct((1, N, 3), jnp.int32),
            jax.ShapeDtypeStruct((1, N, 48), jnp.float32),
        ],
    )(xyz1t_b, xyz2_b)


SC_Q = 16
SC_NW = 32
SC_PER_W = N // SC_NW


def _sc_gather_body(table, idx3, wcat, out,
                    idx_vs, w_vs, rows_vs, out_vs, s_is, s_ws, s_gs, s_os):
    wid = lax.axis_index("s") * 2 + lax.axis_index("c")
    nsteps = SC_PER_W // SC_Q

    def qbase(j):
        return wid * SC_PER_W + j * SC_Q

    def start_a(j, slot):
        pltpu.async_copy(idx3.at[pl.ds(qbase(j) * 3, 3 * SC_Q)],
                         idx_vs[slot], s_is[slot])
        pltpu.async_copy(wcat.at[pl.ds(qbase(j), SC_Q)], w_vs[slot], s_ws[slot])

    def wait_a(j, slot):
        pltpu.make_async_copy(idx3.at[pl.ds(qbase(j) * 3, 3 * SC_Q)],
                              idx_vs[slot], s_is[slot]).wait()
        pltpu.make_async_copy(wcat.at[pl.ds(qbase(j), SC_Q)],
                              w_vs[slot], s_ws[slot]).wait()

    def start_b(slot):
        pltpu.async_copy(table.at[idx_vs[slot]], rows_vs[slot], s_gs[slot])

    def wait_b(slot):
        pltpu.make_async_copy(table.at[idx_vs[slot]], rows_vs[slot],
                              s_gs[slot]).wait()

    def start_o(j, oslot):
        pltpu.async_copy(out_vs[oslot], out.at[pl.ds(qbase(j), SC_Q)],
                         s_os[oslot])

    def wait_o(j, oslot):
        pltpu.make_async_copy(out_vs[oslot], out.at[pl.ds(qbase(j), SC_Q)],
                              s_os[oslot]).wait()

    def compute(j, slot, oslot):
        rv = rows_vs[slot]
        wv = w_vs[slot]
        ov = out_vs[oslot]
        for r in range(SC_Q):
            w0 = wv[r, 0:16]
            w1 = wv[r, 16:32]
            w2 = wv[r, 32:48]
            for c in range(D2 // 16):
                sl = pl.ds(c * 16, 16)
                ov[r, sl] = (w0 * rv[3 * r, sl]
                             + w1 * rv[3 * r + 1, sl]
                             + w2 * rv[3 * r + 2, sl])
        start_o(j, oslot)

    for t in range(4):
        start_a(t, t)
    wait_a(0, 0)
    start_b(0)
    wait_a(1, 1)
    start_b(1)

    def iteration(i, carry):
        j = 4 * i
        for t in range(4):
            jt = j + t
            wait_b(t)
            g = jt + 2

            @pl.when(g < nsteps)
            def _(g=g, t=t):
                wait_a(g, (t + 2) % 4)
                start_b((t + 2) % 4)

            @pl.when(jt >= 2)
            def _(jt=jt, t=t):
                wait_o(jt - 2, t % 2)
            compute(jt, t, t % 2)
            p = jt + 4

            @pl.when(p < nsteps)
            def _(p=p, t=t):
                start_a(p, t)
        return carry

    lax.fori_loop(0, nsteps // 4, iteration, 0)
    wait_o(nsteps - 2, 0)
    wait_o(nsteps - 1, 1)


def _run_sc_gather(table, idx3, wcat):
    mesh = plsc.VectorSubcoreMesh(core_axis_name="c", subcore_axis_name="s")
    fn = pl.kernel(
        _sc_gather_body,
        out_type=jax.ShapeDtypeStruct((N, D2), jnp.float32),
        mesh=mesh,
        scratch_types=[
            [pltpu.VMEM((3 * SC_Q,), jnp.int32) for _ in range(4)],
            [pltpu.VMEM((SC_Q, 48), jnp.float32) for _ in range(4)],
            [pltpu.VMEM((3 * SC_Q, D2), jnp.float32) for _ in range(4)],
            [pltpu.VMEM((SC_Q, D2), jnp.float32) for _ in range(2)],
            [pltpu.SemaphoreType.DMA for _ in range(4)],
            [pltpu.SemaphoreType.DMA for _ in range(4)],
            [pltpu.SemaphoreType.DMA for _ in range(4)],
            [pltpu.SemaphoreType.DMA for _ in range(2)],
        ],
    )
    return fn(table, idx3, wcat)


TNM = 512


def _layer0_body(p1_ref, it_ref, wa_ref, wb_ref, b_ref, y_ref, st_ref):
    step = pl.program_id(0)
    p1 = p1_ref[0]
    it = it_ref[0]
    y = (lax.dot_general(p1, wa_ref[...], (((0,), (0,)), ((), ())),
                         preferred_element_type=jnp.float32)
         + jnp.dot(it, wb_ref[...], preferred_element_type=jnp.float32)
         + b_ref[...])
    y_ref[0] = y

    @pl.when(step == 0)
    def _():
        st_ref[...] = jnp.zeros_like(st_ref)

    s0 = jnp.sum(y, axis=0, keepdims=True)
    s1 = jnp.sum(y * y, axis=0, keepdims=True)
    st_ref[0:1, :] += s0
    st_ref[1:2, :] += s1


def _run_layer0(p1_b, interp_b, wa, wb, b0r):
    return pl.pallas_call(
        _layer0_body,
        grid=(N // TNM,),
        in_specs=[
            pl.BlockSpec((1, D1, TNM), lambda n: (0, 0, n)),
            pl.BlockSpec((1, TNM, D2), lambda n: (0, n, 0)),
            pl.BlockSpec((D1, C0), lambda n: (0, 0)),
            pl.BlockSpec((D2, C0), lambda n: (0, 0)),
            pl.BlockSpec((1, C0), lambda n: (0, 0)),
        ],
        out_specs=[
            pl.BlockSpec((1, TNM, C0), lambda n: (0, n, 0)),
            pl.BlockSpec((8, C0), lambda n: (0, 0)),
        ],
        out_shape=[
            jax.ShapeDtypeStruct((1, N, C0), jnp.float32),
            jax.ShapeDtypeStruct((8, C0), jnp.float32),
        ],
    )(p1_b, interp_b, wa, wb, b0r)


def _layer1_body(y0_ref, a_ref, c_ref, w_ref, b_ref, y_ref, st_ref):
    step = pl.program_id(0)
    h = jnp.maximum(y0_ref[0] * a_ref[...] + c_ref[...], 0.0)
    y = jnp.dot(h, w_ref[...], preferred_element_type=jnp.float32) + b_ref[...]
    y_ref[0] = y

    @pl.when(step == 0)
    def _():
        st_ref[...] = jnp.zeros_like(st_ref)

    st_ref[0:1, :] += jnp.sum(y, axis=0, keepdims=True)
    st_ref[1:2, :] += jnp.sum(y * y, axis=0, keepdims=True)


def _run_layer1(y0t_b, a0, c0, w1t, b1r):
    return pl.pallas_call(
        _layer1_body,
        grid=(N // TNM,),
        in_specs=[
            pl.BlockSpec((1, TNM, C0), lambda n: (0, n, 0)),
            pl.BlockSpec((1, C0), lambda n: (0, 0)),
            pl.BlockSpec((1, C0), lambda n: (0, 0)),
            pl.BlockSpec((C0, C1), lambda n: (0, 0)),
            pl.BlockSpec((1, C1), lambda n: (0, 0)),
        ],
        out_specs=[
            pl.BlockSpec((1, TNM, C1), lambda n: (0, n, 0)),
            pl.BlockSpec((8, C1), lambda n: (0, 0)),
        ],
        out_shape=[
            jax.ShapeDtypeStruct((1, N, C1), jnp.float32),
            jax.ShapeDtypeStruct((8, C1), jnp.float32),
        ],
    )(y0t_b, a0, c0, w1t, b1r)


def _final_body(y1_ref, a_ref, c_ref, o_ref):
    res = jnp.maximum(y1_ref[0] * a_ref[...] + c_ref[...], 0.0)
    o_ref[0] = jnp.transpose(res)


def _run_final(y1t_b, a1, c1):
    return pl.pallas_call(
        _final_body,
        grid=(N // TNM,),
        in_specs=[
            pl.BlockSpec((1, TNM, C1), lambda n: (0, n, 0)),
            pl.BlockSpec((1, C1), lambda n: (0, 0)),
            pl.BlockSpec((1, C1), lambda n: (0, 0)),
        ],
        out_specs=pl.BlockSpec((1, C1, TNM), lambda n: (0, 0, n)),
        out_shape=jax.ShapeDtypeStruct((1, C1, N), jnp.float32),
    )(y1t_b, a1, c1)


def _bn_coeffs(stats, g, beta):
    mean = stats[0, :] / BN_COUNT
    var = stats[1, :] / BN_COUNT - mean * mean
    a = g / jnp.sqrt(var + 1e-5)
    c = beta - mean * a
    return a[None, :], c[None, :]


@jax.jit
def kernel(xyz1, xyz2, points1, points2, W0, b0, g0, beta0, W1, b1, g1, beta1):
    xyz1t = jnp.transpose(xyz1, (0, 2, 1))
    table = jnp.transpose(points2, (0, 2, 1)).reshape(B * S, D2)
    wa = jnp.transpose(W0[:, :D1])
    wb = jnp.transpose(W0[:, D1:])
    b0r = b0[None, :]

    y0ts, st0s = [], []
    for b in range(B):
        idx_b, wcat_b = _run_topk(xyz1t[b:b + 1], xyz2[b:b + 1], b * S)
        interp_b = _run_sc_gather(table, idx_b.reshape(N * 3),
                                  wcat_b.reshape(N, 48))
        y0t_b, st0_b = _run_layer0(points1[b:b + 1],
                                   interp_b.reshape(1, N, D2), wa, wb, b0r)
        y0ts.append(y0t_b)
        st0s.append(st0_b)
    a0, c0 = _bn_coeffs(sum(st0s), g0, beta0)

    w1t = jnp.transpose(W1)
    b1r = b1[None, :]
    y1ts, st1s = [], []
    for b in range(B):
        y1t_b, st1_b = _run_layer1(y0ts[b], a0, c0, w1t, b1r)
        y1ts.append(y1t_b)
        st1s.append(st1_b)
    a1, c1 = _bn_coeffs(sum(st1s), g1, beta1)

    outs = [_run_final(y1ts[b], a1, c1) for b in range(B)]
    return jnp.concatenate(outs, axis=0)


# speedup vs baseline: 1.5225x; 1.2235x over previous
import functools

import jax
import jax.numpy as jnp
from jax import lax
from jax.experimental import pallas as pl
from jax.experimental.pallas import tpu as pltpu
from jax.experimental.pallas import tpu_sc as plsc

B, N, S, D1, D2 = 4, 8192, 2048, 128, 256
C0, C1 = 256, 128
BN_COUNT = B * N


TN1 = 512


def _topk_body(boff, x1_ref, x2_ref, idx_ref, w_ref):
    x1 = x1_ref[0]
    x2 = x2_ref[0]
    n1 = jnp.sum(x1 * x1, axis=1, keepdims=True)
    n2 = jnp.sum(x2 * x2, axis=0, keepdims=True)
    xy = lax.dot_general(x1.astype(jnp.bfloat16), x2.astype(jnp.bfloat16),
                         (((1,), (0,)), ((), ())),
                         preferred_element_type=jnp.float32)
    d = (-2.0 * xy + n1) + n2

    iota = lax.broadcasted_iota(jnp.int32, (TN1, S), 1).astype(jnp.float32)
    big = jnp.float32(jnp.inf)
    fs = jnp.float32(S)
    ds_ = []
    is_ = []
    for _ in range(3):
        m = jnp.min(d, axis=1, keepdims=True)
        i = jnp.min(jnp.where(d == m, iota, fs), axis=1, keepdims=True)
        ds_.append(m)
        is_.append(i)
        d = jnp.where(iota == i, big, d)
    d3 = jnp.concatenate(ds_, axis=1)
    i3 = jnp.concatenate(is_, axis=1).astype(jnp.int32)
    d3 = jnp.maximum(d3, 1e-10)
    recip = 1.0 / d3
    w = recip / jnp.sum(recip, axis=1, keepdims=True)

    idx_ref[0] = i3 + boff
    w_ref[0] = jnp.concatenate(
        [jnp.broadcast_to(w[:, k:k + 1], (TN1, 16)) for k in range(3)], axis=1)


def _run_topk(xyz1t_b, xyz2_b, boff):
    return pl.pallas_call(
        functools.partial(_topk_body, boff),
        grid=(N // TN1,),
        in_specs=[
            pl.BlockSpec((1, TN1, 3), lambda n: (0, n, 0)),
            pl.BlockSpec((1, 3, S), lambda n: (0, 0, 0)),
        ],
        out_specs=[
            pl.BlockSpec((1, TN1, 3), lambda n: (0, n, 0)),
            pl.BlockSpec((1, TN1, 48), lambda n: (0, n, 0)),
        ],
        out_shape=[
            jax.ShapeDtypeStruct((1, N, 3), jnp.int32),
            jax.ShapeDtypeStruct((1, N, 48), jnp.float32),
        ],
    )(xyz1t_b, xyz2_b)


SC_Q = 16
SC_NW = 32
SC_PER_W = N // SC_NW


def _sc_gather_body(table, idx3, wcat, out,
                    idx_vs, w_vs, rows_vs, out_vs, s_is, s_ws, s_gs, s_os):
    wid = lax.axis_index("s") * 2 + lax.axis_index("c")
    nsteps = SC_PER_W // SC_Q

    def qbase(j):
        return wid * SC_PER_W + j * SC_Q

    def start_a(j, slot):
        pltpu.async_copy(idx3.at[pl.ds(qbase(j) * 3, 3 * SC_Q)],
                         idx_vs[slot], s_is[slot])
        pltpu.async_copy(wcat.at[pl.ds(qbase(j), SC_Q)], w_vs[slot], s_ws[slot])

    def wait_a(j, slot):
        pltpu.make_async_copy(idx3.at[pl.ds(qbase(j) * 3, 3 * SC_Q)],
                              idx_vs[slot], s_is[slot]).wait()
        pltpu.make_async_copy(wcat.at[pl.ds(qbase(j), SC_Q)],
                              w_vs[slot], s_ws[slot]).wait()

    def start_b(slot):
        pltpu.async_copy(table.at[idx_vs[slot]], rows_vs[slot], s_gs[slot])

    def wait_b(slot):
        pltpu.make_async_copy(table.at[idx_vs[slot]], rows_vs[slot],
                              s_gs[slot]).wait()

    def start_o(j, oslot):
        pltpu.async_copy(out_vs[oslot], out.at[pl.ds(qbase(j), SC_Q)],
                         s_os[oslot])

    def wait_o(j, oslot):
        pltpu.make_async_copy(out_vs[oslot], out.at[pl.ds(qbase(j), SC_Q)],
                              s_os[oslot]).wait()

    def compute(j, slot, oslot):
        rv = rows_vs[slot]
        wv = w_vs[slot]
        ov = out_vs[oslot]
        for r in range(SC_Q):
            w0 = wv[r, 0:16]
            w1 = wv[r, 16:32]
            w2 = wv[r, 32:48]
            for c in range(D2 // 16):
                sl = pl.ds(c * 16, 16)
                ov[r, sl] = (w0 * rv[3 * r, sl]
                             + w1 * rv[3 * r + 1, sl]
                             + w2 * rv[3 * r + 2, sl])
        start_o(j, oslot)

    for t in range(4):
        start_a(t, t)
    wait_a(0, 0)
    start_b(0)
    wait_a(1, 1)
    start_b(1)

    def iteration(i, carry):
        j = 4 * i
        for t in range(4):
            jt = j + t
            wait_b(t)
            g = jt + 2

            @pl.when(g < nsteps)
            def _(g=g, t=t):
                wait_a(g, (t + 2) % 4)
                start_b((t + 2) % 4)

            @pl.when(jt >= 2)
            def _(jt=jt, t=t):
                wait_o(jt - 2, t % 2)
            compute(jt, t, t % 2)
            p = jt + 4

            @pl.when(p < nsteps)
            def _(p=p, t=t):
                start_a(p, t)
        return carry

    lax.fori_loop(0, nsteps // 4, iteration, 0)
    wait_o(nsteps - 2, 0)
    wait_o(nsteps - 1, 1)


def _run_sc_gather(table, idx3, wcat):
    mesh = plsc.VectorSubcoreMesh(core_axis_name="c", subcore_axis_name="s")
    fn = pl.kernel(
        _sc_gather_body,
        out_type=jax.ShapeDtypeStruct((N, D2), jnp.float32),
        mesh=mesh,
        scratch_types=[
            [pltpu.VMEM((3 * SC_Q,), jnp.int32) for _ in range(4)],
            [pltpu.VMEM((SC_Q, 48), jnp.float32) for _ in range(4)],
            [pltpu.VMEM((3 * SC_Q, D2), jnp.float32) for _ in range(4)],
            [pltpu.VMEM((SC_Q, D2), jnp.float32) for _ in range(2)],
            [pltpu.SemaphoreType.DMA for _ in range(4)],
            [pltpu.SemaphoreType.DMA for _ in range(4)],
            [pltpu.SemaphoreType.DMA for _ in range(4)],
            [pltpu.SemaphoreType.DMA for _ in range(2)],
        ],
    )
    return fn(table, idx3, wcat)


TNM = 512


def _layer0_body(p1_ref, it_ref, wa_ref, wb_ref, b_ref, y_ref, st_ref):
    step = pl.program_id(0)
    p1 = p1_ref[0]
    it = it_ref[0]
    y = (lax.dot_general(p1, wa_ref[...], (((0,), (0,)), ((), ())),
                         preferred_element_type=jnp.float32)
         + jnp.dot(it, wb_ref[...], preferred_element_type=jnp.float32)
         + b_ref[...])
    y_ref[0] = y

    @pl.when(step == 0)
    def _():
        st_ref[...] = jnp.zeros_like(st_ref)

    s0 = jnp.sum(y, axis=0, keepdims=True)
    s1 = jnp.sum(y * y, axis=0, keepdims=True)
    st_ref[0:1, :] += s0
    st_ref[1:2, :] += s1


def _run_layer0(p1_b, interp_b, wa, wb, b0r):
    return pl.pallas_call(
        _layer0_body,
        grid=(N // TNM,),
        in_specs=[
            pl.BlockSpec((1, D1, TNM), lambda n: (0, 0, n)),
            pl.BlockSpec((1, TNM, D2), lambda n: (0, n, 0)),
            pl.BlockSpec((D1, C0), lambda n: (0, 0)),
            pl.BlockSpec((D2, C0), lambda n: (0, 0)),
            pl.BlockSpec((1, C0), lambda n: (0, 0)),
        ],
        out_specs=[
            pl.BlockSpec((1, TNM, C0), lambda n: (0, n, 0)),
            pl.BlockSpec((8, C0), lambda n: (0, 0)),
        ],
        out_shape=[
            jax.ShapeDtypeStruct((1, N, C0), jnp.float32),
            jax.ShapeDtypeStruct((8, C0), jnp.float32),
        ],
    )(p1_b, interp_b, wa, wb, b0r)


def _layer1_body(y0_ref, a_ref, c_ref, w_ref, b_ref, y_ref, st_ref):
    step = pl.program_id(0)
    h = jnp.maximum(y0_ref[0] * a_ref[...] + c_ref[...], 0.0)
    y = jnp.dot(h, w_ref[...], preferred_element_type=jnp.float32) + b_ref[...]
    y_ref[0] = y

    @pl.when(step == 0)
    def _():
        st_ref[...] = jnp.zeros_like(st_ref)

    st_ref[0:1, :] += jnp.sum(y, axis=0, keepdims=True)
    st_ref[1:2, :] += jnp.sum(y * y, axis=0, keepdims=True)


def _run_layer1(y0t_b, a0, c0, w1t, b1r):
    return pl.pallas_call(
        _layer1_body,
        grid=(N // TNM,),
        in_specs=[
            pl.BlockSpec((1, TNM, C0), lambda n: (0, n, 0)),
            pl.BlockSpec((1, C0), lambda n: (0, 0)),
            pl.BlockSpec((1, C0), lambda n: (0, 0)),
            pl.BlockSpec((C0, C1), lambda n: (0, 0)),
            pl.BlockSpec((1, C1), lambda n: (0, 0)),
        ],
        out_specs=[
            pl.BlockSpec((1, TNM, C1), lambda n: (0, n, 0)),
            pl.BlockSpec((8, C1), lambda n: (0, 0)),
        ],
        out_shape=[
            jax.ShapeDtypeStruct((1, N, C1), jnp.float32),
            jax.ShapeDtypeStruct((8, C1), jnp.float32),
        ],
    )(y0t_b, a0, c0, w1t, b1r)


def _final_body(y1_ref, a_ref, c_ref, o_ref):
    res = jnp.maximum(y1_ref[0] * a_ref[...] + c_ref[...], 0.0)
    o_ref[0] = jnp.transpose(res)


def _run_final(y1t_b, a1, c1):
    return pl.pallas_call(
        _final_body,
        grid=(N // TNM,),
        in_specs=[
            pl.BlockSpec((1, TNM, C1), lambda n: (0, n, 0)),
            pl.BlockSpec((1, C1), lambda n: (0, 0)),
            pl.BlockSpec((1, C1), lambda n: (0, 0)),
        ],
        out_specs=pl.BlockSpec((1, C1, TNM), lambda n: (0, 0, n)),
        out_shape=jax.ShapeDtypeStruct((1, C1, N), jnp.float32),
    )(y1t_b, a1, c1)


def _bn_coeffs(stats, g, beta):
    mean = stats[0, :] / BN_COUNT
    var = stats[1, :] / BN_COUNT - mean * mean
    a = g / jnp.sqrt(var + 1e-5)
    c = beta - mean * a
    return a[None, :], c[None, :]


@jax.jit
def kernel(xyz1, xyz2, points1, points2, W0, b0, g0, beta0, W1, b1, g1, beta1):
    xyz1t = jnp.transpose(xyz1, (0, 2, 1))
    table = jnp.transpose(points2, (0, 2, 1)).reshape(B * S, D2)
    wa = jnp.transpose(W0[:, :D1])
    wb = jnp.transpose(W0[:, D1:])
    b0r = b0[None, :]

    y0ts, st0s = [], []
    for b in range(B):
        idx_b, wcat_b = _run_topk(xyz1t[b:b + 1], xyz2[b:b + 1], b * S)
        interp_b = _run_sc_gather(table, idx_b.reshape(N * 3),
                                  wcat_b.reshape(N, 48))
        y0t_b, st0_b = _run_layer0(points1[b:b + 1],
                                   interp_b.reshape(1, N, D2), wa, wb, b0r)
        y0ts.append(y0t_b)
        st0s.append(st0_b)
    a0, c0 = _bn_coeffs(sum(st0s), g0, beta0)

    w1t = jnp.transpose(W1)
    b1r = b1[None, :]
    y1ts, st1s = [], []
    for b in range(B):
        y1t_b, st1_b = _run_layer1(y0ts[b], a0, c0, w1t, b1r)
        y1ts.append(y1t_b)
        st1s.append(st1_b)
    a1, c1 = _bn_coeffs(sum(st1s), g1, beta1)

    outs = [_run_final(y1ts[b], a1, c1) for b in range(B)]
    return jnp.concatenate(outs, axis=0)


# speedup vs baseline: 1.5238x; 1.0009x over previous
import functools

import jax
import jax.numpy as jnp
from jax import lax
from jax.experimental import pallas as pl
from jax.experimental.pallas import tpu as pltpu
from jax.experimental.pallas import tpu_sc as plsc

B, N, S, D1, D2 = 4, 8192, 2048, 128, 256
C0, C1 = 256, 128
BN_COUNT = B * N


TN1 = 512


def _topk_body(boff, x1_ref, x2_ref, idx_ref, w_ref):
    x1 = x1_ref[0]
    x2 = x2_ref[0]
    n1 = jnp.sum(x1 * x1, axis=1, keepdims=True)
    n2 = jnp.sum(x2 * x2, axis=0, keepdims=True)
    xy = lax.dot_general(x1.astype(jnp.bfloat16), x2.astype(jnp.bfloat16),
                         (((1,), (0,)), ((), ())),
                         preferred_element_type=jnp.float32)
    d = (-2.0 * xy + n1) + n2

    iota = lax.broadcasted_iota(jnp.int32, (TN1, S), 1).astype(jnp.float32)
    big = jnp.float32(jnp.inf)
    fs = jnp.float32(S)
    ds_ = []
    is_ = []
    for k in range(3):
        m = jnp.min(d, axis=1, keepdims=True)
        i = jnp.min(jnp.where(d == m, iota, fs), axis=1, keepdims=True)
        ds_.append(m)
        is_.append(i)
        if k < 2:
            d = jnp.where(iota == i, big, d)
    d3 = jnp.concatenate(ds_, axis=1)
    i3 = jnp.concatenate(is_, axis=1).astype(jnp.int32)
    d3 = jnp.maximum(d3, 1e-10)
    recip = 1.0 / d3
    w = recip / jnp.sum(recip, axis=1, keepdims=True)

    idx_ref[0] = i3 + boff
    w_ref[0] = jnp.concatenate(
        [jnp.broadcast_to(w[:, k:k + 1], (TN1, 16)) for k in range(3)], axis=1)


def _run_topk(xyz1t_b, xyz2_b, boff):
    return pl.pallas_call(
        functools.partial(_topk_body, boff),
        grid=(N // TN1,),
        in_specs=[
            pl.BlockSpec((1, TN1, 3), lambda n: (0, n, 0)),
            pl.BlockSpec((1, 3, S), lambda n: (0, 0, 0)),
        ],
        out_specs=[
            pl.BlockSpec((1, TN1, 3), lambda n: (0, n, 0)),
            pl.BlockSpec((1, TN1, 48), lambda n: (0, n, 0)),
        ],
        out_shape=[
            jax.ShapeDtypeStruct((1, N, 3), jnp.int32),
            jax.ShapeDtypeStruct((1, N, 48), jnp.float32),
        ],
    )(xyz1t_b, xyz2_b)


SC_Q = 16
SC_NW = 32
SC_PER_W = N // SC_NW


def _sc_gather_body(table, idx3, wcat, out,
                    idx_vs, w_vs, rows_vs, out_vs, s_is, s_ws, s_gs, s_os):
    wid = lax.axis_index("s") * 2 + lax.axis_index("c")
    nsteps = SC_PER_W // SC_Q

    def qbase(j):
        return wid * SC_PER_W + j * SC_Q

    def start_a(j, slot):
        pltpu.async_copy(idx3.at[pl.ds(qbase(j) * 3, 3 * SC_Q)],
                         idx_vs[slot], s_is[slot])
        pltpu.async_copy(wcat.at[pl.ds(qbase(j), SC_Q)], w_vs[slot], s_ws[slot])

    def wait_a(j, slot):
        pltpu.make_async_copy(idx3.at[pl.ds(qbase(j) * 3, 3 * SC_Q)],
                              idx_vs[slot], s_is[slot]).wait()
        pltpu.make_async_copy(wcat.at[pl.ds(qbase(j), SC_Q)],
                              w_vs[slot], s_ws[slot]).wait()

    def start_b(slot):
        pltpu.async_copy(table.at[idx_vs[slot]], rows_vs[slot], s_gs[slot])

    def wait_b(slot):
        pltpu.make_async_copy(table.at[idx_vs[slot]], rows_vs[slot],
                              s_gs[slot]).wait()

    def start_o(j, oslot):
        pltpu.async_copy(out_vs[oslot], out.at[pl.ds(qbase(j), SC_Q)],
                         s_os[oslot])

    def wait_o(j, oslot):
        pltpu.make_async_copy(out_vs[oslot], out.at[pl.ds(qbase(j), SC_Q)],
                              s_os[oslot]).wait()

    def compute(j, slot, oslot):
        rv = rows_vs[slot]
        wv = w_vs[slot]
        ov = out_vs[oslot]
        for r in range(SC_Q):
            w0 = wv[r, 0:16]
            w1 = wv[r, 16:32]
            w2 = wv[r, 32:48]
            for c in range(D2 // 16):
                sl = pl.ds(c * 16, 16)
                ov[r, sl] = (w0 * rv[3 * r, sl]
                             + w1 * rv[3 * r + 1, sl]
                             + w2 * rv[3 * r + 2, sl])
        start_o(j, oslot)

    for t in range(4):
        start_a(t, t)
    wait_a(0, 0)
    start_b(0)
    wait_a(1, 1)
    start_b(1)

    def iteration(i, carry):
        j = 4 * i
        for t in range(4):
            jt = j + t
            wait_b(t)
            g = jt + 2

            @pl.when(g < nsteps)
            def _(g=g, t=t):
                wait_a(g, (t + 2) % 4)
                start_b((t + 2) % 4)

            @pl.when(jt >= 2)
            def _(jt=jt, t=t):
                wait_o(jt - 2, t % 2)
            compute(jt, t, t % 2)
            p = jt + 4

            @pl.when(p < nsteps)
            def _(p=p, t=t):
                start_a(p, t)
        return carry

    lax.fori_loop(0, nsteps // 4, iteration, 0)
    wait_o(nsteps - 2, 0)
    wait_o(nsteps - 1, 1)


def _run_sc_gather(table, idx3, wcat):
    mesh = plsc.VectorSubcoreMesh(core_axis_name="c", subcore_axis_name="s")
    fn = pl.kernel(
        _sc_gather_body,
        out_type=jax.ShapeDtypeStruct((N, D2), jnp.float32),
        mesh=mesh,
        scratch_types=[
            [pltpu.VMEM((3 * SC_Q,), jnp.int32) for _ in range(4)],
            [pltpu.VMEM((SC_Q, 48), jnp.float32) for _ in range(4)],
            [pltpu.VMEM((3 * SC_Q, D2), jnp.float32) for _ in range(4)],
            [pltpu.VMEM((SC_Q, D2), jnp.float32) for _ in range(2)],
            [pltpu.SemaphoreType.DMA for _ in range(4)],
            [pltpu.SemaphoreType.DMA for _ in range(4)],
            [pltpu.SemaphoreType.DMA for _ in range(4)],
            [pltpu.SemaphoreType.DMA for _ in range(2)],
        ],
    )
    return fn(table, idx3, wcat)


TNM = 512


def _layer0_body(p1_ref, it_ref, wa_ref, wb_ref, b_ref, y_ref, st_ref):
    step = pl.program_id(0)
    p1 = p1_ref[0]
    it = it_ref[0]
    y = (lax.dot_general(p1, wa_ref[...], (((0,), (0,)), ((), ())),
                         preferred_element_type=jnp.float32)
         + jnp.dot(it, wb_ref[...], preferred_element_type=jnp.float32)
         + b_ref[...])
    y_ref[0] = y

    @pl.when(step == 0)
    def _():
        st_ref[...] = jnp.zeros_like(st_ref)

    s0 = jnp.sum(y, axis=0, keepdims=True)
    s1 = jnp.sum(y * y, axis=0, keepdims=True)
    st_ref[0:1, :] += s0
    st_ref[1:2, :] += s1


def _run_layer0(p1_b, interp_b, wa, wb, b0r):
    return pl.pallas_call(
        _layer0_body,
        grid=(N // TNM,),
        in_specs=[
            pl.BlockSpec((1, D1, TNM), lambda n: (0, 0, n)),
            pl.BlockSpec((1, TNM, D2), lambda n: (0, n, 0)),
            pl.BlockSpec((D1, C0), lambda n: (0, 0)),
            pl.BlockSpec((D2, C0), lambda n: (0, 0)),
            pl.BlockSpec((1, C0), lambda n: (0, 0)),
        ],
        out_specs=[
            pl.BlockSpec((1, TNM, C0), lambda n: (0, n, 0)),
            pl.BlockSpec((8, C0), lambda n: (0, 0)),
        ],
        out_shape=[
            jax.ShapeDtypeStruct((1, N, C0), jnp.float32),
            jax.ShapeDtypeStruct((8, C0), jnp.float32),
        ],
    )(p1_b, interp_b, wa, wb, b0r)


def _layer1_body(y0_ref, a_ref, c_ref, w_ref, b_ref, y_ref, st_ref):
    step = pl.program_id(0)
    h = jnp.maximum(y0_ref[0] * a_ref[...] + c_ref[...], 0.0)
    y = jnp.dot(h, w_ref[...], preferred_element_type=jnp.float32) + b_ref[...]
    y_ref[0] = y

    @pl.when(step == 0)
    def _():
        st_ref[...] = jnp.zeros_like(st_ref)

    st_ref[0:1, :] += jnp.sum(y, axis=0, keepdims=True)
    st_ref[1:2, :] += jnp.sum(y * y, axis=0, keepdims=True)


def _run_layer1(y0t_b, a0, c0, w1t, b1r):
    return pl.pallas_call(
        _layer1_body,
        grid=(N // TNM,),
        in_specs=[
            pl.BlockSpec((1, TNM, C0), lambda n: (0, n, 0)),
            pl.BlockSpec((1, C0), lambda n: (0, 0)),
            pl.BlockSpec((1, C0), lambda n: (0, 0)),
            pl.BlockSpec((C0, C1), lambda n: (0, 0)),
            pl.BlockSpec((1, C1), lambda n: (0, 0)),
        ],
        out_specs=[
            pl.BlockSpec((1, TNM, C1), lambda n: (0, n, 0)),
            pl.BlockSpec((8, C1), lambda n: (0, 0)),
        ],
        out_shape=[
            jax.ShapeDtypeStruct((1, N, C1), jnp.float32),
            jax.ShapeDtypeStruct((8, C1), jnp.float32),
        ],
    )(y0t_b, a0, c0, w1t, b1r)


def _final_body(y1_ref, a_ref, c_ref, o_ref):
    res = jnp.maximum(y1_ref[0] * a_ref[...] + c_ref[...], 0.0)
    o_ref[0] = jnp.transpose(res)


def _run_final(y1t_b, a1, c1):
    return pl.pallas_call(
        _final_body,
        grid=(N // TNM,),
        in_specs=[
            pl.BlockSpec((1, TNM, C1), lambda n: (0, n, 0)),
            pl.BlockSpec((1, C1), lambda n: (0, 0)),
            pl.BlockSpec((1, C1), lambda n: (0, 0)),
        ],
        out_specs=pl.BlockSpec((1, C1, TNM), lambda n: (0, 0, n)),
        out_shape=jax.ShapeDtypeStruct((1, C1, N), jnp.float32),
    )(y1t_b, a1, c1)


def _bn_coeffs(stats, g, beta):
    mean = stats[0, :] / BN_COUNT
    var = stats[1, :] / BN_COUNT - mean * mean
    a = g / jnp.sqrt(var + 1e-5)
    c = beta - mean * a
    return a[None, :], c[None, :]


@jax.jit
def kernel(xyz1, xyz2, points1, points2, W0, b0, g0, beta0, W1, b1, g1, beta1):
    xyz1t = jnp.transpose(xyz1, (0, 2, 1))
    table = jnp.transpose(points2, (0, 2, 1)).reshape(B * S, D2)
    wa = jnp.transpose(W0[:, :D1])
    wb = jnp.transpose(W0[:, D1:])
    b0r = b0[None, :]

    y0ts, st0s = [], []
    for b in range(B):
        idx_b, wcat_b = _run_topk(xyz1t[b:b + 1], xyz2[b:b + 1], b * S)
        interp_b = _run_sc_gather(table, idx_b.reshape(N * 3),
                                  wcat_b.reshape(N, 48))
        y0t_b, st0_b = _run_layer0(points1[b:b + 1],
                                   interp_b.reshape(1, N, D2), wa, wb, b0r)
        y0ts.append(y0t_b)
        st0s.append(st0_b)
    a0, c0 = _bn_coeffs(sum(st0s), g0, beta0)

    w1t = jnp.transpose(W1)
    b1r = b1[None, :]
    y1ts, st1s = [], []
    for b in range(B):
        y1t_b, st1_b = _run_layer1(y0ts[b], a0, c0, w1t, b1r)
        y1ts.append(y1t_b)
        st1s.append(st1_b)
    a1, c1 = _bn_coeffs(sum(st1s), g1, beta1)

    outs = [_run_final(y1ts[b], a1, c1) for b in range(B)]
    return jnp.concatenate(outs, axis=0)
